# baseline (device time: 42745 ns/iter reference)
import jax
import jax.numpy as jnp
from jax import lax
from jax.experimental import pallas as pl
from jax.experimental.pallas import tpu as pltpu

N_DEV = 8
B, SQ, SKV, HQ, DH, DM = 2, 512, 512, 64, 64, 768
H_PER = HQ // N_DEV
CHUNK = SQ // N_DEV
BLK = 4 * CHUNK
WINDOW = 128
_COMM = True


def kernel(x, Wq, K_ext, V_ext, Wo):
    my = lax.axis_index("i")

    K = lax.dynamic_slice_in_dim(K_ext, my * H_PER, H_PER, axis=2)
    V = lax.dynamic_slice_in_dim(V_ext, my * H_PER, H_PER, axis=2)
    K = jnp.transpose(K, (2, 0, 1, 3)).astype(jnp.bfloat16)
    V = jnp.transpose(V, (2, 0, 1, 3)).astype(jnp.bfloat16)
    Wq_r = (Wq * 0.125).astype(jnp.bfloat16)
    Wo_b = Wo.astype(jnp.bfloat16)
    x_b = x.astype(jnp.bfloat16)

    def body(x_ref, wq_ref, k_ref, v_ref, wo_ref, out_ref,
             rs_buf, ss_rs, ag_ss, rs_sem, ag_sem, dummy_sem):
        my_i = lax.axis_index("i")

        if _COMM:
            barrier = pltpu.get_barrier_semaphore()
            for k in range(1, N_DEV):
                pl.semaphore_signal(
                    barrier, inc=1,
                    device_id=((my_i + k) % N_DEV,),
                    device_id_type=pl.DeviceIdType.MESH,
                )
            pl.semaphore_wait(barrier, N_DEV - 1)

        def compute_rows(rb, nrows):
            rr = lax.broadcasted_iota(jnp.int32, (nrows, SKV), 0)
            cc = lax.broadcasted_iota(jnp.int32, (nrows, SKV), 1)
            bias = jnp.where(
                jnp.abs((rr + rb) - cc) <= WINDOW, 0.0, -1e9
            ).astype(jnp.float32)
            for b in range(B):
                xb = x_ref[b, pl.ds(rb, nrows), :]
                q_all = jnp.dot(xb, wq_ref[...],
                                preferred_element_type=jnp.float32
                                ).astype(jnp.bfloat16)
                ctxs = []
                for h in range(H_PER):
                    q = q_all[:, h * DH:(h + 1) * DH]
                    s = lax.dot_general(
                        q, k_ref[h, b], (((1,), (1,)), ((), ())),
                        preferred_element_type=jnp.float32,
                    ) + bias
                    w = jnp.exp(s)
                    denom = jnp.sum(w, axis=-1, keepdims=True)
                    ctx = jnp.dot(w.astype(jnp.bfloat16), v_ref[h, b],
                                  preferred_element_type=jnp.float32)
                    ctxs.append((ctx / denom).astype(jnp.bfloat16))
                ctx_all = jnp.concatenate(ctxs, axis=1)
                o = jnp.dot(ctx_all, wo_ref[...],
                            preferred_element_type=jnp.float32)
                out_ref[b, pl.ds(rb, nrows), :] = o.astype(jnp.bfloat16)

        def send_chunk(dest, sem_slot, descs):
            slot = (my_i - dest) % N_DEV
            rdma = pltpu.make_async_remote_copy(
                src_ref=out_ref.at[:, pl.ds(dest * CHUNK, CHUNK), :],
                dst_ref=rs_buf.at[slot],
                send_sem=ss_rs.at[sem_slot],
                recv_sem=rs_sem,
                device_id=(dest,),
                device_id_type=pl.DeviceIdType.MESH,
            )
            rdma.start()
            descs.append(rdma)

        mb = my_i // 4
        hm = (my_i % 4) // 2
        rs_descs = []
        ob = 1 - mb
        compute_rows(ob * BLK, BLK)
        if _COMM:
            for j in range(4):
                send_chunk(4 * ob + j, j, rs_descs)
        first = mb * BLK + (1 - hm) * 128
        compute_rows(first, 128)
        if _COMM:
            c0 = 4 * mb + 2 * (1 - hm)
            send_chunk(c0, 4, rs_descs)
            send_chunk(c0 + 1, 5, rs_descs)
        compute_rows(mb * BLK + hm * 128, 128)
        if _COMM:
            send_chunk(my_i ^ 1, 6, rs_descs)

        my_rows = (slice(None), pl.ds(my_i * CHUNK, CHUNK), slice(None))
        if _COMM:
            for k in range(1, N_DEV):
                recv = pltpu.make_async_remote_copy(
                    src_ref=rs_buf.at[k], dst_ref=rs_buf.at[k],
                    send_sem=dummy_sem, recv_sem=rs_sem,
                    device_id=(my_i,), device_id_type=pl.DeviceIdType.MESH,
                )
                recv.wait_recv()
            acc = out_ref[my_rows].astype(jnp.float32)
            for k in range(1, N_DEV):
                acc = acc + rs_buf[k].astype(jnp.float32)
            out_ref[my_rows] = acc.astype(jnp.bfloat16)

            ag_descs = []
            for k in range(1, N_DEV):
                rdma = pltpu.make_async_remote_copy(
                    src_ref=out_ref.at[my_rows],
                    dst_ref=out_ref.at[my_rows],
                    send_sem=ag_ss.at[k - 1],
                    recv_sem=ag_sem,
                    device_id=((my_i + k) % N_DEV,),
                    device_id_type=pl.DeviceIdType.MESH,
                )
                rdma.start()
                ag_descs.append(rdma)
            for r in rs_descs:
                r.wait_send()
            for k in range(1, N_DEV):
                recv = pltpu.make_async_remote_copy(
                    src_ref=out_ref.at[:, pl.ds(k * CHUNK, CHUNK), :],
                    dst_ref=out_ref.at[:, pl.ds(k * CHUNK, CHUNK), :],
                    send_sem=dummy_sem, recv_sem=ag_sem,
                    device_id=(my_i,), device_id_type=pl.DeviceIdType.MESH,
                )
                recv.wait_recv()
            for r in ag_descs:
                r.wait_send()

    return pl.pallas_call(
        body,
        out_shape=jax.ShapeDtypeStruct((B, SQ, DM), jnp.bfloat16),
        in_specs=[pl.BlockSpec(memory_space=pltpu.VMEM)] * 5,
        out_specs=pl.BlockSpec(memory_space=pltpu.VMEM),
        scratch_shapes=[
            pltpu.VMEM((N_DEV, B, CHUNK, DM), jnp.bfloat16),
            pltpu.SemaphoreType.DMA((7,)),
            pltpu.SemaphoreType.DMA((7,)),
            pltpu.SemaphoreType.DMA,
            pltpu.SemaphoreType.DMA,
            pltpu.SemaphoreType.DMA,
        ],
        compiler_params=pltpu.CompilerParams(
            collective_id=0 if _COMM else None
        ),
    )(x_b, Wq_r, K, V, Wo_b)


# device time: 40790 ns/iter; 1.0479x vs baseline; 1.0479x over previous
import jax
import jax.numpy as jnp
from jax import lax
from jax.experimental import pallas as pl
from jax.experimental.pallas import tpu as pltpu

N_DEV = 8
B, SQ, SKV, HQ, DH, DM = 2, 512, 512, 64, 64, 768
H_PER = HQ // N_DEV
CHUNK = SQ // N_DEV
BLK = 4 * CHUNK
WINDOW = 128
_COMM = True


def kernel(x, Wq, K_ext, V_ext, Wo):
    my = lax.axis_index("i")

    K = lax.dynamic_slice_in_dim(K_ext, my * H_PER, H_PER, axis=2)
    V = lax.dynamic_slice_in_dim(V_ext, my * H_PER, H_PER, axis=2)
    K = jnp.transpose(K, (2, 0, 1, 3)).astype(jnp.bfloat16)
    V = jnp.transpose(V, (2, 0, 1, 3)).astype(jnp.bfloat16)
    Wq_r = (Wq * 0.125).astype(jnp.bfloat16)
    Wo_b = Wo.astype(jnp.bfloat16)
    x_b = x.astype(jnp.bfloat16)

    def body(x_ref, wq_ref, k_ref, v_ref, wo_ref, out_ref,
             rs_buf, ss_rs, ag_ss, rs_sems, ag_sems, dummy_sem):
        my_i = lax.axis_index("i")

        if _COMM:
            barrier = pltpu.get_barrier_semaphore()
            for k in range(1, N_DEV):
                pl.semaphore_signal(
                    barrier, inc=1,
                    device_id=((my_i + k) % N_DEV,),
                    device_id_type=pl.DeviceIdType.MESH,
                )
            pl.semaphore_wait(barrier, N_DEV - 1)

        def compute_rows(b, rb, nrows):
            rr = lax.broadcasted_iota(jnp.int32, (nrows, SKV), 0)
            cc = lax.broadcasted_iota(jnp.int32, (nrows, SKV), 1)
            bias = jnp.where(
                jnp.abs((rr + rb) - cc) <= WINDOW, 0.0, -1e9
            ).astype(jnp.float32)
            xb = x_ref[b, pl.ds(rb, nrows), :]
            q_all = jnp.dot(xb, wq_ref[...],
                            preferred_element_type=jnp.float32
                            ).astype(jnp.bfloat16)
            ctxs = []
            for h in range(H_PER):
                q = q_all[:, h * DH:(h + 1) * DH]
                s = lax.dot_general(
                    q, k_ref[h, b], (((1,), (1,)), ((), ())),
                    preferred_element_type=jnp.float32,
                ) + bias
                w = jnp.exp(s)
                denom = jnp.sum(w, axis=-1, keepdims=True)
                ctx = jnp.dot(w.astype(jnp.bfloat16), v_ref[h, b],
                              preferred_element_type=jnp.float32)
                ctxs.append((ctx / denom).astype(jnp.bfloat16))
            ctx_all = jnp.concatenate(ctxs, axis=1)
            o = jnp.dot(ctx_all, wo_ref[...],
                        preferred_element_type=jnp.float32)
            out_ref[b, pl.ds(rb, nrows), :] = o.astype(jnp.bfloat16)

        def send_chunk(b, dest, sem_slot, descs):
            slot = (my_i - dest) % N_DEV
            rdma = pltpu.make_async_remote_copy(
                src_ref=out_ref.at[b, pl.ds(dest * CHUNK, CHUNK), :],
                dst_ref=rs_buf.at[b, slot],
                send_sem=ss_rs.at[sem_slot],
                recv_sem=rs_sems.at[b],
                device_id=(dest,),
                device_id_type=pl.DeviceIdType.MESH,
            )
            rdma.start()
            descs.append(rdma)

        mb = my_i // 4
        hm = (my_i % 4) // 2
        ob = 1 - mb
        c_first = 4 * mb + 2 * (1 - hm)
        rs_descs, ag_descs = [], []

        def phase_compute(b, base_slot):
            compute_rows(b, ob * BLK, BLK)
            if _COMM:
                for j in range(4):
                    send_chunk(b, 4 * ob + j, base_slot + j, rs_descs)
            compute_rows(b, mb * BLK + (1 - hm) * 128, 128)
            if _COMM:
                send_chunk(b, c_first, base_slot + 4, rs_descs)
                send_chunk(b, c_first + 1, base_slot + 5, rs_descs)
            compute_rows(b, mb * BLK + hm * 128, 128)
            if _COMM:
                send_chunk(b, my_i ^ 1, base_slot + 6, rs_descs)

        def phase_finalize(b, base_slot):
            my_rows = (b, pl.ds(my_i * CHUNK, CHUNK), slice(None))
            for k in range(1, N_DEV):
                recv = pltpu.make_async_remote_copy(
                    src_ref=rs_buf.at[b, k], dst_ref=rs_buf.at[b, k],
                    send_sem=dummy_sem, recv_sem=rs_sems.at[b],
                    device_id=(my_i,), device_id_type=pl.DeviceIdType.MESH,
                )
                recv.wait_recv()
            acc = out_ref[my_rows].astype(jnp.float32)
            for k in range(1, N_DEV):
                acc = acc + rs_buf[b, k].astype(jnp.float32)
            out_ref[my_rows] = acc.astype(jnp.bfloat16)
            for k in range(1, N_DEV):
                rdma = pltpu.make_async_remote_copy(
                    src_ref=out_ref.at[my_rows],
                    dst_ref=out_ref.at[my_rows],
                    send_sem=ag_ss.at[base_slot + k - 1],
                    recv_sem=ag_sems.at[b],
                    device_id=((my_i + k) % N_DEV,),
                    device_id_type=pl.DeviceIdType.MESH,
                )
                rdma.start()
                ag_descs.append(rdma)

        phase_compute(0, 0)
        compute_rows(1, ob * BLK, BLK)
        if _COMM:
            for j in range(4):
                send_chunk(1, 4 * ob + j, 7 + j, rs_descs)
            phase_finalize(0, 0)
        compute_rows(1, mb * BLK + (1 - hm) * 128, 128)
        if _COMM:
            send_chunk(1, c_first, 11, rs_descs)
            send_chunk(1, c_first + 1, 12, rs_descs)
        compute_rows(1, mb * BLK + hm * 128, 128)
        if _COMM:
            send_chunk(1, my_i ^ 1, 13, rs_descs)
            phase_finalize(1, 7)

            for r in rs_descs:
                r.wait_send()
            for b in range(B):
                for k in range(1, N_DEV):
                    recv = pltpu.make_async_remote_copy(
                        src_ref=out_ref.at[b, pl.ds(k * CHUNK, CHUNK), :],
                        dst_ref=out_ref.at[b, pl.ds(k * CHUNK, CHUNK), :],
                        send_sem=dummy_sem, recv_sem=ag_sems.at[b],
                        device_id=(my_i,),
                        device_id_type=pl.DeviceIdType.MESH,
                    )
                    recv.wait_recv()
            for r in ag_descs:
                r.wait_send()

    return pl.pallas_call(
        body,
        out_shape=jax.ShapeDtypeStruct((B, SQ, DM), jnp.bfloat16),
        in_specs=[pl.BlockSpec(memory_space=pltpu.VMEM)] * 5,
        out_specs=pl.BlockSpec(memory_space=pltpu.VMEM),
        scratch_shapes=[
            pltpu.VMEM((B, N_DEV, CHUNK, DM), jnp.bfloat16),
            pltpu.SemaphoreType.DMA((14,)),
            pltpu.SemaphoreType.DMA((14,)),
            pltpu.SemaphoreType.DMA((2,)),
            pltpu.SemaphoreType.DMA((2,)),
            pltpu.SemaphoreType.DMA,
        ],
        compiler_params=pltpu.CompilerParams(
            collective_id=0 if _COMM else None
        ),
    )(x_b, Wq_r, K, V, Wo_b)
